# baseline (speedup 1.0000x reference)
import functools

import jax
import jax.numpy as jnp
from jax import lax
from jax.experimental import pallas as pl
from jax.experimental.pallas import tpu as pltpu
from jax.experimental.pallas import tpu_sc as plsc


def _mlp_kernel(xc_ref, e_ref, b1t_ref, g_ref, b2t_ref, out_ref):
    bh = e_ref.shape[1]
    gw = g_ref.shape[0]
    hist = g_ref.shape[1] // gw
    n_groups = bh // gw

    h = jnp.maximum(
        jnp.dot(xc_ref[...], e_ref[...], preferred_element_type=jnp.float32)
        + b1t_ref[0][None, :],
        0.0,
    )

    for g in range(n_groups):
        og = jnp.dot(h[:, g * gw:(g + 1) * gw], g_ref[...],
                     preferred_element_type=jnp.float32)
        for t in range(hist):
            lo = t * bh + g * gw
            out_ref[:, lo:lo + gw] = (
                og[:, t * gw:(t + 1) * gw] + b2t_ref[0][None, lo:lo + gw]
            ).astype(jnp.bfloat16)


def _tc_mlp_bf16(input, W1, b1, W2, b2):
    B, N, in_dim = input.shape
    hid = W1.shape[1]
    hist = W2.shape[1] // hid

    nb = 400

    xc = jnp.transpose(input, (1, 2, 0)).reshape(N, in_dim * B)
    eye_b = jnp.eye(B, dtype=jnp.float32)
    e_mat = jnp.einsum('bc,dk->dbck', eye_b, W1).reshape(in_dim * B, B * hid)
    b1t = jnp.tile(b1, B).reshape(1, B * hid)
    w2r = W2.reshape(hid, hist, hid)
    eye4 = jnp.eye(4, dtype=jnp.float32)
    g_mat = jnp.einsum('bc,ktj->bktcj', eye4, w2r).reshape(4 * hid,
                                                           hist * 4 * hid)
    b2t = jnp.tile(b2.reshape(hist, 1, hid), (1, B, 1)).reshape(1,
                                                                hist * B * hid)

    return pl.pallas_call(
        _mlp_kernel,
        grid=(N // nb,),
        in_specs=[
            pl.BlockSpec((nb, in_dim * B), lambda i: (i, 0)),
            pl.BlockSpec((in_dim * B, B * hid), lambda i: (0, 0)),
            pl.BlockSpec((1, B * hid), lambda i: (0, 0)),
            pl.BlockSpec((4 * hid, hist * 4 * hid), lambda i: (0, 0)),
            pl.BlockSpec((1, hist * B * hid), lambda i: (0, 0)),
        ],
        out_specs=pl.BlockSpec((nb, hist * B * hid), lambda i: (i, 0)),
        out_shape=jax.ShapeDtypeStruct((N, hist * B * hid), jnp.bfloat16),
        compiler_params=pltpu.CompilerParams(
            dimension_semantics=("parallel",),
        ),
    )(xc, e_mat, b1t, g_mat, b2t)


_NW = 32
_WCHUNK = 32000
_UNROLL = 8


def _sc_widen_body(in_hbm, out_hbm, in_v, out_v):
    words = in_hbm.shape[0]
    per_w = words // _NW
    n_chunks = per_w // _WCHUNK
    groups = _WCHUNK // 16
    wid = lax.axis_index("s") * 2 + lax.axis_index("c")
    base = wid * per_w
    iota16 = lax.iota(jnp.int32, 16)
    himask = jnp.int32(-65536)

    def chunk_body(c, carry):
        off = base + c * _WCHUNK
        pltpu.sync_copy(in_hbm.at[pl.ds(off, _WCHUNK)], in_v)

        def grp_body(i, carry2):
            for u in range(_UNROLL):
                g = i * _UNROLL + u
                w32 = in_v[pl.ds(g * 16, 16)]
                evens = lax.shift_left(w32, jnp.int32(16))
                odds = lax.bitwise_and(w32, himask)
                eidx = g * 32 + 2 * iota16
                plsc.store_scatter(out_v, [eidx], evens)
                plsc.store_scatter(out_v, [eidx + 1], odds)
            return carry2

        lax.fori_loop(0, groups // _UNROLL, grp_body, 0)
        pltpu.sync_copy(out_v, out_hbm.at[pl.ds(off * 2, _WCHUNK * 2)])
        return carry

    lax.fori_loop(0, n_chunks, chunk_body, 0)


def _sc_widen(y16_flat):
    total = y16_flat.shape[0]
    y32 = lax.bitcast_convert_type(
        y16_flat.reshape(total // 2, 2), jnp.int32)
    mesh = plsc.VectorSubcoreMesh(core_axis_name="c", subcore_axis_name="s",
                                  num_cores=2, num_subcores=16)
    fn = functools.partial(
        pl.kernel,
        mesh=mesh,
        out_type=jax.ShapeDtypeStruct((total,), jnp.int32),
        scratch_types=[
            pltpu.VMEM((_WCHUNK,), jnp.int32),
            pltpu.VMEM((_WCHUNK * 2,), jnp.int32),
        ],
        compiler_params=pltpu.CompilerParams(needs_layout_passes=False),
    )(_sc_widen_body)
    return lax.bitcast_convert_type(fn(y32), jnp.float32)


def kernel(input, W1, b1, W2, b2):
    B, N, in_dim = input.shape
    hid = W1.shape[1]
    hist = W2.shape[1] // hid

    y16 = _tc_mlp_bf16(input, W1, b1, W2, b2)
    out = _sc_widen(y16.reshape(N * hist * B * hid))
    return out.reshape(N, hist, B, hid)
